# Initial kernel scaffold; baseline (speedup 1.0000x reference)
import functools

import jax
import jax.numpy as jnp
from jax import lax
from jax.experimental import pallas as pl
from jax.experimental.pallas import tpu as pltpu
from jax.experimental.pallas import tpu_sc as plsc

_B, _T = 1024, 200
_N = _B * _T
_EMB = 64
_OUT_D = 3 * _EMB + 32
_EPS = 1e-5

_NC, _NS = 2, 16
_NW = _NC * _NS
_ROWS_PER_W = _N // _NW
_IDXROW = 128
_CH_ROWS = 5
_CH = _CH_ROWS * _IDXROW
_NCH = _ROWS_PER_W // _CH


def _sc_gather_kernel(i0_hbm, i1_hbm, i2_hbm, t0_hbm, t1_hbm, t2_hbm,
                      out_hbm, idx_v, rows_v, sem):
    wid = lax.axis_index("s") * _NC + lax.axis_index("c")
    base = wid * _ROWS_PER_W
    irow_base = wid * (_ROWS_PER_W // _IDXROW)
    for tbl, (ih, th) in enumerate(((i0_hbm, t0_hbm), (i1_hbm, t1_hbm),
                                    (i2_hbm, t2_hbm))):
        col = tbl * _EMB

        @pl.loop(0, _NCH)
        def _(c):
            r0 = base + c * _CH
            ir0 = irow_base + c * _CH_ROWS
            pltpu.sync_copy(ih.at[pl.ds(ir0, _CH_ROWS)], idx_v)
            cps = []
            for j in range(_CH_ROWS):
                cps.append(pltpu.async_copy(
                    th.at[idx_v.at[j]],
                    rows_v.at[pl.ds(j * _IDXROW, _IDXROW)], sem))
            for cp in cps:
                cp.wait()
            pltpu.sync_copy(rows_v,
                            out_hbm.at[pl.ds(r0, _CH), pl.ds(col, _EMB)])


def _sc_gather(idx0, idx1, idx2, tab0, tab1, tab2):
    mesh = plsc.VectorSubcoreMesh(core_axis_name="c", subcore_axis_name="s")
    k = pl.kernel(
        _sc_gather_kernel,
        out_type=jax.ShapeDtypeStruct((_N, _OUT_D), jnp.float32),
        mesh=mesh,
        scratch_types=[
            pltpu.VMEM((_CH_ROWS, _IDXROW), jnp.int32),
            pltpu.VMEM((_CH, _EMB), jnp.float32),
            pltpu.SemaphoreType.DMA,
        ],
    )
    return k(idx0, idx1, idx2, tab0, tab1, tab2)


_BB = 32
_NSTEP = _B // _BB


def _tc_numeric_kernel(alias_ref, nf0_ref, nf1_ref, seq_ref, wb_ref,
                       scal_ref, out_ref):
    del alias_ref
    i = pl.program_id(0)
    lens_full = seq_ref[...]
    iota_t = lax.broadcasted_iota(jnp.int32, (_B, _T), 1)
    mfull = (iota_t < lens_full).astype(jnp.float32)
    cnt = jnp.maximum(jnp.sum(mfull), 1.0)

    r0 = i * _BB
    lens = seq_ref[pl.ds(r0, _BB), :]
    iota_b = lax.broadcasted_iota(jnp.int32, (_BB, _T), 1)
    mask = iota_b < lens

    outs = []
    for f, ref in enumerate((nf0_ref, nf1_ref)):
        xf = ref[...]
        s1 = jnp.sum(xf * mfull)
        s2 = jnp.sum(xf * xf * mfull)
        mean = s1 / cnt
        var = jnp.maximum(s2 / cnt - mean * mean, 0.0)
        rstd = lax.rsqrt(var + _EPS)
        gamma = scal_ref[2 * f]
        beta = scal_ref[2 * f + 1]
        x = ref[pl.ds(r0, _BB), :]
        xn = (x - mean) * (rstd * gamma) + beta
        y = jnp.where(mask, xn, x)
        w = wb_ref[2 * f, :]
        b = wb_ref[2 * f + 1, :]
        outs.append(y[:, :, None] * w[None, None, :] + b[None, None, :])
    out_ref[...] = jnp.concatenate(outs, axis=-1)


def _tc_numeric(out3d, nf0, nf1, seq2d, wb, scal):
    return pl.pallas_call(
        _tc_numeric_kernel,
        grid=(_NSTEP,),
        in_specs=[
            pl.BlockSpec(memory_space=pltpu.ANY),
            pl.BlockSpec((_B, _T), lambda i: (0, 0)),
            pl.BlockSpec((_B, _T), lambda i: (0, 0)),
            pl.BlockSpec((_B, 1), lambda i: (0, 0)),
            pl.BlockSpec((4, 16), lambda i: (0, 0)),
            pl.BlockSpec(memory_space=pltpu.SMEM),
        ],
        out_specs=pl.BlockSpec((_BB, _T, 32), lambda i: (i, 0, 6)),
        out_shape=jax.ShapeDtypeStruct((_B, _T, _OUT_D), jnp.float32),
        input_output_aliases={0: 0},
    )(out3d, nf0, nf1, seq2d, wb, scal)


def kernel(emb_feat_0, emb_feat_1, emb_feat_2, num_feat_0, num_feat_1,
           event_time, seq_lens, emb_table_0, emb_table_1, emb_table_2,
           bn_gamma_0, bn_beta_0, bn_gamma_1, bn_beta_1,
           lin_w_0, lin_b_0, lin_w_1, lin_b_1):
    idx0 = emb_feat_0.astype(jnp.int32).reshape(_N // _IDXROW, _IDXROW)
    idx1 = emb_feat_1.astype(jnp.int32).reshape(_N // _IDXROW, _IDXROW)
    idx2 = emb_feat_2.astype(jnp.int32).reshape(_N // _IDXROW, _IDXROW)

    cat = _sc_gather(idx0, idx1, idx2, emb_table_0, emb_table_1, emb_table_2)
    out3d = cat.reshape(_B, _T, _OUT_D)

    seq2d = seq_lens.astype(jnp.int32).reshape(_B, 1)
    wb = jnp.stack([lin_w_0[0].astype(jnp.float32),
                    lin_b_0.astype(jnp.float32),
                    lin_w_1[0].astype(jnp.float32),
                    lin_b_1.astype(jnp.float32)], axis=0)
    scal = jnp.stack([bn_gamma_0.astype(jnp.float32),
                      bn_beta_0.astype(jnp.float32),
                      bn_gamma_1.astype(jnp.float32),
                      bn_beta_1.astype(jnp.float32)])

    out = _tc_numeric(out3d, num_feat_0.astype(jnp.float32),
                      num_feat_1.astype(jnp.float32), seq2d, wb, scal)
    return out, event_time.astype(jnp.float32)


# speedup vs baseline: 2.4678x; 2.4678x over previous
import jax
import jax.numpy as jnp
from jax import lax
from jax.experimental import pallas as pl
from jax.experimental.pallas import tpu as pltpu
from jax.experimental.pallas import tpu_sc as plsc

_B, _T = 1024, 200
_N = _B * _T
_EMB = 64
_OUT_D = 3 * _EMB + 32
_EPS = 1e-5

_NC, _NS = 2, 16
_NW = _NC * _NS
_ROWS_PER_W = _N // _NW
_IDXROW = 128
_CH_ROWS = 5
_CH = _CH_ROWS * _IDXROW
_NCH = _ROWS_PER_W // _CH


def _sc_gather_kernel(i0_hbm, i1_hbm, i2_hbm, t0_hbm, t1_hbm, t2_hbm,
                      o0_hbm, o1_hbm, o2_hbm, idx_v, rows_v, sem):
    wid = lax.axis_index("s") * _NC + lax.axis_index("c")
    base = wid * _ROWS_PER_W

    for ih, th, oh in ((i0_hbm, t0_hbm, o0_hbm), (i1_hbm, t1_hbm, o1_hbm),
                       (i2_hbm, t2_hbm, o2_hbm)):
        @pl.loop(0, _NCH)
        def _(c):
            r0 = base + c * _CH
            pltpu.sync_copy(ih.at[pl.ds(r0, _CH)], idx_v)
            cps = []
            for j in range(_CH_ROWS):
                cps.append(pltpu.async_copy(
                    th.at[idx_v.at[pl.ds(j * _IDXROW, _IDXROW)]],
                    rows_v.at[pl.ds(j * _IDXROW, _IDXROW)], sem))
            for cp in cps:
                cp.wait()
            pltpu.sync_copy(rows_v, oh.at[pl.ds(r0, _CH)])


def _sc_gather(idx0, idx1, idx2, tab0, tab1, tab2):
    mesh = plsc.VectorSubcoreMesh(core_axis_name="c", subcore_axis_name="s",
                                  num_cores=_NC, num_subcores=_NS)
    row_ty = jax.ShapeDtypeStruct((_N, _EMB), jnp.float32)
    k = pl.kernel(
        _sc_gather_kernel,
        out_type=(row_ty, row_ty, row_ty),
        mesh=mesh,
        scratch_types=[
            pltpu.VMEM((_CH,), jnp.int32),
            pltpu.VMEM((_CH, _EMB), jnp.float32),
            pltpu.SemaphoreType.DMA,
        ],
        compiler_params=pltpu.CompilerParams(use_tc_tiling_on_sc=False),
    )
    return k(idx0, idx1, idx2, tab0, tab1, tab2)


_BB = 32
_NSTEP = _B // _BB


def _tc_fuse_kernel(c0_ref, c1_ref, c2_ref, nf0_ref, nf1_ref, seq_ref,
                    wb_ref, scal_ref, out_ref):
    i = pl.program_id(0)
    lens_full = seq_ref[...]
    iota_t = lax.broadcasted_iota(jnp.int32, (_B, _T), 1)
    mfull = (iota_t < lens_full).astype(jnp.float32)
    cnt = jnp.maximum(jnp.sum(mfull), 1.0)

    r0 = i * _BB
    lens = seq_ref[pl.ds(r0, _BB), :]
    iota_b = lax.broadcasted_iota(jnp.int32, (_BB, _T), 1)
    mask = iota_b < lens

    pieces = [c0_ref[...], c1_ref[...], c2_ref[...]]
    for f, ref in enumerate((nf0_ref, nf1_ref)):
        xf = ref[...]
        s1 = jnp.sum(xf * mfull)
        s2 = jnp.sum(xf * xf * mfull)
        mean = s1 / cnt
        var = jnp.maximum(s2 / cnt - mean * mean, 0.0)
        rstd = lax.rsqrt(var + _EPS)
        gamma = scal_ref[2 * f]
        beta = scal_ref[2 * f + 1]
        x = ref[pl.ds(r0, _BB), :]
        xn = (x - mean) * (rstd * gamma) + beta
        y = jnp.where(mask, xn, x)
        w = wb_ref[2 * f, :]
        b = wb_ref[2 * f + 1, :]
        pieces.append(y[:, :, None] * w[None, None, :] + b[None, None, :])
    out_ref[...] = jnp.concatenate(pieces, axis=-1)


def _tc_fuse(c0, c1, c2, nf0, nf1, seq2d, wb, scal):
    cat_spec = pl.BlockSpec((_BB, _T, _EMB), lambda i: (i, 0, 0))
    return pl.pallas_call(
        _tc_fuse_kernel,
        grid=(_NSTEP,),
        in_specs=[
            cat_spec, cat_spec, cat_spec,
            pl.BlockSpec((_B, _T), lambda i: (0, 0)),
            pl.BlockSpec((_B, _T), lambda i: (0, 0)),
            pl.BlockSpec((_B, 1), lambda i: (0, 0)),
            pl.BlockSpec((4, 16), lambda i: (0, 0)),
            pl.BlockSpec(memory_space=pltpu.SMEM),
        ],
        out_specs=pl.BlockSpec((_BB, _T, _OUT_D), lambda i: (i, 0, 0)),
        out_shape=jax.ShapeDtypeStruct((_B, _T, _OUT_D), jnp.float32),
    )(c0, c1, c2, nf0, nf1, seq2d, wb, scal)


def kernel(emb_feat_0, emb_feat_1, emb_feat_2, num_feat_0, num_feat_1,
           event_time, seq_lens, emb_table_0, emb_table_1, emb_table_2,
           bn_gamma_0, bn_beta_0, bn_gamma_1, bn_beta_1,
           lin_w_0, lin_b_0, lin_w_1, lin_b_1):
    idx0 = emb_feat_0.astype(jnp.int32).reshape(_N)
    idx1 = emb_feat_1.astype(jnp.int32).reshape(_N)
    idx2 = emb_feat_2.astype(jnp.int32).reshape(_N)

    c0, c1, c2 = _sc_gather(idx0, idx1, idx2, emb_table_0, emb_table_1,
                            emb_table_2)

    seq2d = seq_lens.astype(jnp.int32).reshape(_B, 1)
    wb = jnp.stack([lin_w_0[0].astype(jnp.float32),
                    lin_b_0.astype(jnp.float32),
                    lin_w_1[0].astype(jnp.float32),
                    lin_b_1.astype(jnp.float32)], axis=0)
    scal = jnp.stack([bn_gamma_0.astype(jnp.float32),
                      bn_beta_0.astype(jnp.float32),
                      bn_gamma_1.astype(jnp.float32),
                      bn_beta_1.astype(jnp.float32)])

    out = _tc_fuse(c0.reshape(_B, _T, _EMB), c1.reshape(_B, _T, _EMB),
                   c2.reshape(_B, _T, _EMB),
                   num_feat_0.astype(jnp.float32),
                   num_feat_1.astype(jnp.float32), seq2d, wb, scal)
    return out, event_time.astype(jnp.float32)


# speedup vs baseline: 3.4465x; 1.3966x over previous
import jax
import jax.numpy as jnp
from jax import lax
from jax.experimental import pallas as pl
from jax.experimental.pallas import tpu as pltpu
from jax.experimental.pallas import tpu_sc as plsc

_B, _T = 1024, 200
_N = _B * _T
_EMB = 64
_OUT_D = 3 * _EMB + 32
_EPS = 1e-5

_NC, _NS = 2, 16
_NW = _NC * _NS
_ROWS_PER_W = _N // _NW
_IDXROW = 128
_CH_ROWS = 5
_CH = _CH_ROWS * _IDXROW
_NCH = _ROWS_PER_W // _CH


def _sc_gather_kernel(i0_hbm, i1_hbm, i2_hbm, t0_hbm, t1_hbm, t2_hbm,
                      o01_hbm, o2_hbm, idx_v, rows_v, sem):
    wid = lax.axis_index("s") * _NC + lax.axis_index("c")
    base = wid * _ROWS_PER_W

    for ih, th, oh, col in ((i0_hbm, t0_hbm, o01_hbm, 0),
                            (i1_hbm, t1_hbm, o01_hbm, _EMB),
                            (i2_hbm, t2_hbm, o2_hbm, 0)):
        @pl.loop(0, _NCH)
        def _(c):
            r0 = base + c * _CH
            pltpu.sync_copy(ih.at[pl.ds(r0, _CH)], idx_v)
            cps = []
            for j in range(_CH_ROWS):
                cps.append(pltpu.async_copy(
                    th.at[idx_v.at[pl.ds(j * _IDXROW, _IDXROW)]],
                    rows_v.at[pl.ds(j * _IDXROW, _IDXROW)], sem))
            for cp in cps:
                cp.wait()
            pltpu.sync_copy(rows_v,
                            oh.at[pl.ds(r0, _CH), pl.ds(col, _EMB)])


def _sc_gather(idx0, idx1, idx2, tab0, tab1, tab2):
    mesh = plsc.VectorSubcoreMesh(core_axis_name="c", subcore_axis_name="s",
                                  num_cores=_NC, num_subcores=_NS)
    wide_ty = jax.ShapeDtypeStruct((_N, 2 * _EMB), jnp.float32)
    k = pl.kernel(
        _sc_gather_kernel,
        out_type=(wide_ty, wide_ty),
        mesh=mesh,
        scratch_types=[
            pltpu.VMEM((_CH,), jnp.int32),
            pltpu.VMEM((_CH, _EMB), jnp.float32),
            pltpu.SemaphoreType.DMA,
        ],
        compiler_params=pltpu.CompilerParams(use_tc_tiling_on_sc=False),
    )
    return k(idx0, idx1, idx2, tab0, tab1, tab2)


_BB = 32
_NSTEP = _B // _BB


def _tc_fuse_kernel(c01_ref, c2_ref, nf0_ref, nf1_ref, seq_ref,
                    wb_ref, scal_ref, out_ref):
    i = pl.program_id(0)
    lens_full = seq_ref[...]
    iota_t = lax.broadcasted_iota(jnp.int32, (_B, _T), 1)
    mfull = (iota_t < lens_full).astype(jnp.float32)
    cnt = jnp.maximum(jnp.sum(mfull), 1.0)

    r0 = i * _BB
    lens = seq_ref[pl.ds(r0, _BB), :]
    iota_b = lax.broadcasted_iota(jnp.int32, (_BB, _T), 1)
    mask = iota_b < lens

    pieces = [c01_ref[...], c2_ref[..., :_EMB]]
    for f, ref in enumerate((nf0_ref, nf1_ref)):
        xf = ref[...]
        s1 = jnp.sum(xf * mfull)
        s2 = jnp.sum(xf * xf * mfull)
        mean = s1 / cnt
        var = jnp.maximum(s2 / cnt - mean * mean, 0.0)
        rstd = lax.rsqrt(var + _EPS)
        gamma = scal_ref[2 * f]
        beta = scal_ref[2 * f + 1]
        x = ref[pl.ds(r0, _BB), :]
        xn = (x - mean) * (rstd * gamma) + beta
        y = jnp.where(mask, xn, x)
        w = wb_ref[2 * f, :]
        b = wb_ref[2 * f + 1, :]
        pieces.append(y[:, :, None] * w[None, None, :] + b[None, None, :])
    out_ref[...] = jnp.concatenate(pieces, axis=-1)


def _tc_fuse(c01, c2, nf0, nf1, seq2d, wb, scal):
    cat_spec = pl.BlockSpec((_BB, _T, 2 * _EMB), lambda i: (i, 0, 0))
    return pl.pallas_call(
        _tc_fuse_kernel,
        grid=(_NSTEP,),
        in_specs=[
            cat_spec, cat_spec,
            pl.BlockSpec((_B, _T), lambda i: (0, 0)),
            pl.BlockSpec((_B, _T), lambda i: (0, 0)),
            pl.BlockSpec((_B, 1), lambda i: (0, 0)),
            pl.BlockSpec((4, 16), lambda i: (0, 0)),
            pl.BlockSpec(memory_space=pltpu.SMEM),
        ],
        out_specs=pl.BlockSpec((_BB, _T, _OUT_D), lambda i: (i, 0, 0)),
        out_shape=jax.ShapeDtypeStruct((_B, _T, _OUT_D), jnp.float32),
    )(c01, c2, nf0, nf1, seq2d, wb, scal)


def kernel(emb_feat_0, emb_feat_1, emb_feat_2, num_feat_0, num_feat_1,
           event_time, seq_lens, emb_table_0, emb_table_1, emb_table_2,
           bn_gamma_0, bn_beta_0, bn_gamma_1, bn_beta_1,
           lin_w_0, lin_b_0, lin_w_1, lin_b_1):
    idx0 = emb_feat_0.astype(jnp.int32).reshape(_N)
    idx1 = emb_feat_1.astype(jnp.int32).reshape(_N)
    idx2 = emb_feat_2.astype(jnp.int32).reshape(_N)

    c01, c2 = _sc_gather(idx0, idx1, idx2, emb_table_0, emb_table_1,
                         emb_table_2)

    seq2d = seq_lens.astype(jnp.int32).reshape(_B, 1)
    wb = jnp.stack([lin_w_0[0].astype(jnp.float32),
                    lin_b_0.astype(jnp.float32),
                    lin_w_1[0].astype(jnp.float32),
                    lin_b_1.astype(jnp.float32)], axis=0)
    scal = jnp.stack([bn_gamma_0.astype(jnp.float32),
                      bn_beta_0.astype(jnp.float32),
                      bn_gamma_1.astype(jnp.float32),
                      bn_beta_1.astype(jnp.float32)])

    out = _tc_fuse(c01.reshape(_B, _T, 2 * _EMB),
                   c2.reshape(_B, _T, 2 * _EMB),
                   num_feat_0.astype(jnp.float32),
                   num_feat_1.astype(jnp.float32), seq2d, wb, scal)
    return out, event_time.astype(jnp.float32)


# speedup vs baseline: 5.1570x; 1.4963x over previous
import jax
import jax.numpy as jnp
from jax import lax
from jax.experimental import pallas as pl
from jax.experimental.pallas import tpu as pltpu
from jax.experimental.pallas import tpu_sc as plsc

_B, _T = 1024, 200
_N = _B * _T
_EMB = 64
_OUT_D = 3 * _EMB + 32
_EPS = 1e-5

_NC, _NS = 2, 16
_NW = _NC * _NS
_ROWS_PER_W = _N // _NW
_IDXROW = 128
_CH_ROWS = 5
_CH = _CH_ROWS * _IDXROW
_NCH = _ROWS_PER_W // _CH


def _sc_gather_kernel(i0_hbm, i1_hbm, i2_hbm, t0_hbm, t1_hbm, t2_hbm,
                      o01_hbm, o2_hbm, idx_v, rows_v, sem):
    wid = lax.axis_index("s") * _NC + lax.axis_index("c")
    base = wid * _ROWS_PER_W

    for ih, th, oh, col in ((i0_hbm, t0_hbm, o01_hbm, 0),
                            (i1_hbm, t1_hbm, o01_hbm, _EMB),
                            (i2_hbm, t2_hbm, o2_hbm, 0)):
        @pl.loop(0, _NCH)
        def _(c):
            r0 = base + c * _CH
            pltpu.sync_copy(ih.at[pl.ds(r0, _CH)], idx_v)
            cps = []
            for j in range(_CH_ROWS):
                cps.append(pltpu.async_copy(
                    th.at[idx_v.at[pl.ds(j * _IDXROW, _IDXROW)]],
                    rows_v.at[pl.ds(j * _IDXROW, _IDXROW)], sem))
            for cp in cps:
                cp.wait()
            pltpu.sync_copy(rows_v,
                            oh.at[pl.ds(r0, _CH), pl.ds(col, _EMB)])


def _sc_gather(idx0, idx1, idx2, tab0, tab1, tab2):
    mesh = plsc.VectorSubcoreMesh(core_axis_name="c", subcore_axis_name="s",
                                  num_cores=_NC, num_subcores=_NS)
    wide_ty = jax.ShapeDtypeStruct((_N, 2 * _EMB), jnp.float32)
    k = pl.kernel(
        _sc_gather_kernel,
        out_type=(wide_ty, wide_ty),
        mesh=mesh,
        scratch_types=[
            pltpu.VMEM((_CH,), jnp.int32),
            pltpu.VMEM((_CH, _EMB), jnp.float32),
            pltpu.SemaphoreType.DMA,
        ],
        compiler_params=pltpu.CompilerParams(use_tc_tiling_on_sc=False),
    )
    return k(idx0, idx1, idx2, tab0, tab1, tab2)


_TB = 8
_NSTEP = _T // _TB


def _tc_fuse_kernel(c01_ref, c2_ref, nf0_ref, nf1_ref, seq_ref,
                    wb_ref, scal_ref, out_ref):
    i = pl.program_id(0)
    seq = seq_ref[...]
    iota_t = lax.broadcasted_iota(jnp.int32, (_T, _B), 0)
    mfull = (iota_t < seq).astype(jnp.float32)
    cnt = jnp.maximum(jnp.sum(mfull), 1.0)

    t0 = i * _TB
    iota_b = lax.broadcasted_iota(jnp.int32, (_TB, _B), 0) + t0
    mask = iota_b < seq

    c01 = jnp.swapaxes(c01_ref[...], 1, 2)
    c2 = jnp.swapaxes(c2_ref[...], 1, 2)[:, :_EMB, :]
    pieces = [c01, c2]
    for f, ref in enumerate((nf0_ref, nf1_ref)):
        xf = ref[...]
        s1 = jnp.sum(xf * mfull)
        s2 = jnp.sum(xf * xf * mfull)
        mean = s1 / cnt
        var = jnp.maximum(s2 / cnt - mean * mean, 0.0)
        rstd = lax.rsqrt(var + _EPS)
        gamma = scal_ref[2 * f]
        beta = scal_ref[2 * f + 1]
        x = ref[pl.ds(t0, _TB), :]
        xn = (x - mean) * (rstd * gamma) + beta
        y = jnp.where(mask, xn, x)
        w = wb_ref[2 * f, :]
        b = wb_ref[2 * f + 1, :]
        pieces.append(y[:, None, :] * w[None, :, None] + b[None, :, None])
    out_ref[...] = jnp.concatenate(pieces, axis=1)


def _tc_fuse(c01, c2, nf0, nf1, seq2d, wb, scal):
    cat_spec = pl.BlockSpec((_TB, _B, 2 * _EMB), lambda i: (i, 0, 0))
    return pl.pallas_call(
        _tc_fuse_kernel,
        grid=(_NSTEP,),
        in_specs=[
            cat_spec, cat_spec,
            pl.BlockSpec((_T, _B), lambda i: (0, 0)),
            pl.BlockSpec((_T, _B), lambda i: (0, 0)),
            pl.BlockSpec((1, _B), lambda i: (0, 0)),
            pl.BlockSpec((4, 16), lambda i: (0, 0)),
            pl.BlockSpec(memory_space=pltpu.SMEM),
        ],
        out_specs=pl.BlockSpec((_TB, _OUT_D, _B), lambda i: (i, 0, 0)),
        out_shape=jax.ShapeDtypeStruct((_T, _OUT_D, _B), jnp.float32),
    )(c01, c2, nf0, nf1, seq2d, wb, scal)


def kernel(emb_feat_0, emb_feat_1, emb_feat_2, num_feat_0, num_feat_1,
           event_time, seq_lens, emb_table_0, emb_table_1, emb_table_2,
           bn_gamma_0, bn_beta_0, bn_gamma_1, bn_beta_1,
           lin_w_0, lin_b_0, lin_w_1, lin_b_1):
    idx0 = jnp.transpose(emb_feat_0.astype(jnp.int32), (1, 0)).reshape(_N)
    idx1 = jnp.transpose(emb_feat_1.astype(jnp.int32), (1, 0)).reshape(_N)
    idx2 = jnp.transpose(emb_feat_2.astype(jnp.int32), (1, 0)).reshape(_N)

    c01, c2 = _sc_gather(idx0, idx1, idx2, emb_table_0, emb_table_1,
                         emb_table_2)

    seq2d = seq_lens.astype(jnp.int32).reshape(1, _B)
    wb = jnp.stack([lin_w_0[0].astype(jnp.float32),
                    lin_b_0.astype(jnp.float32),
                    lin_w_1[0].astype(jnp.float32),
                    lin_b_1.astype(jnp.float32)], axis=0)
    scal = jnp.stack([bn_gamma_0.astype(jnp.float32),
                      bn_beta_0.astype(jnp.float32),
                      bn_gamma_1.astype(jnp.float32),
                      bn_beta_1.astype(jnp.float32)])

    nf0_t = jnp.transpose(num_feat_0.astype(jnp.float32), (1, 0))
    nf1_t = jnp.transpose(num_feat_1.astype(jnp.float32), (1, 0))
    out_t = _tc_fuse(c01.reshape(_T, _B, 2 * _EMB),
                     c2.reshape(_T, _B, 2 * _EMB),
                     nf0_t, nf1_t, seq2d, wb, scal)
    out = jnp.transpose(out_t, (2, 0, 1))
    return out, event_time.astype(jnp.float32)
